# Initial kernel scaffold; baseline (speedup 1.0000x reference)
import jax
import jax.numpy as jnp
from jax.experimental import pallas as pl


def kernel(x, edge_index, W1, b1, W2, b2):
    raise NotImplementedError("write your pallas kernel here")


# speedup vs baseline: 24.1672x; 24.1672x over previous
import functools

import jax
import jax.numpy as jnp
from jax import lax
from jax.experimental import pallas as pl
from jax.experimental.pallas import tpu as pltpu
from jax.experimental.pallas import tpu_sc as plsc

N = 10000
E = 320000
IN_DIM = 128
HID_DIM = 128
OUT_DIM = 64

NC = 2
NS = 16
NW = NC * NS
EPW = E // NW
CH = 125
NCHUNK = EPW // CH
ROWS_PT = N // NS

_SELU_ALPHA = 1.6732632423543772
_SELU_SCALE = 1.0507009873554805


def _make_sc_scatter(D):
  mesh = plsc.VectorSubcoreMesh(core_axis_name="c", subcore_axis_name="s")

  @functools.partial(
      pl.kernel,
      out_type=jax.ShapeDtypeStruct((NC, N, D), jnp.float32),
      mesh=mesh,
      compiler_params=pltpu.CompilerParams(use_tc_tiling_on_sc=False),
      scratch_types=[
          pltpu.VMEM((NCHUNK, CH), jnp.int32),
          pltpu.VMEM((NCHUNK, CH), jnp.int32),
          pltpu.VMEM((CH, D), jnp.float32),
          pltpu.VMEM_SHARED((N, D), jnp.float32),
          pltpu.SemaphoreType.DMA,
      ],
  )
  def sc_scatter(table_hbm, src_hbm, dst_hbm, zeros_hbm, out_hbm,
                 src_v, dst_v, buf, acc, sem):
    cid = lax.axis_index("c")
    sid = lax.axis_index("s")
    wid = sid * NC + cid

    @pl.when(sid == 0)
    def _():
      pltpu.sync_copy(zeros_hbm, acc)

    pltpu.sync_copy(src_hbm.at[wid], src_v)
    pltpu.sync_copy(dst_hbm.at[wid], dst_v)
    plsc.subcore_barrier()

    def body(j, carry):
      pltpu.async_copy(table_hbm.at[src_v.at[j]], buf, sem).wait()
      pltpu.sync_copy(buf, acc.at[dst_v.at[j]], add=True)
      return carry

    lax.fori_loop(0, NCHUNK, body, 0, unroll=False)
    plsc.subcore_barrier()

    @pl.when(sid == 0)
    def _():
      pltpu.sync_copy(acc, out_hbm.at[cid])

  return sc_scatter


DEG_W = 16


def _make_sc_degree():
  mesh = plsc.VectorSubcoreMesh(core_axis_name="c", subcore_axis_name="s")

  @functools.partial(
      pl.kernel,
      out_type=jax.ShapeDtypeStruct((NC, N, DEG_W), jnp.float32),
      mesh=mesh,
      compiler_params=pltpu.CompilerParams(use_tc_tiling_on_sc=False),
      scratch_types=[
          pltpu.VMEM((NCHUNK, CH), jnp.int32),
          pltpu.VMEM((CH, DEG_W), jnp.float32),
          pltpu.VMEM_SHARED((N, DEG_W), jnp.float32),
          pltpu.SemaphoreType.DMA,
      ],
  )
  def sc_degree(ones_hbm, dst_hbm, zeros_hbm, out_hbm, dst_v, buf, acc, sem):
    cid = lax.axis_index("c")
    sid = lax.axis_index("s")
    wid = sid * NC + cid

    @pl.when(sid == 0)
    def _():
      pltpu.sync_copy(zeros_hbm, acc)

    pltpu.sync_copy(dst_hbm.at[wid], dst_v)
    pltpu.sync_copy(ones_hbm, buf)
    plsc.subcore_barrier()

    def body(j, carry):
      pltpu.sync_copy(buf, acc.at[dst_v.at[j]], add=True)
      return carry

    lax.fori_loop(0, NCHUNK, body, 0, unroll=False)
    plsc.subcore_barrier()

    @pl.when(sid == 0)
    def _():
      pltpu.sync_copy(acc, out_hbm.at[cid])

  return sc_degree


_ROW_BLK = 1000
_GRID = N // _ROW_BLK


def _dis_from_degp(degp_blk):
  deg = degp_blk[0, :, 0:1] + degp_blk[1, :, 0:1] + 1.0
  return lax.rsqrt(deg)


def _tc_stage1(x_ref, w1_ref, degp_ref, g1_ref):
  dis = _dis_from_degp(degp_ref[...])
  h = jnp.dot(x_ref[...], w1_ref[...], preferred_element_type=jnp.float32)
  g1_ref[...] = h * dis


def _tc_stage2(s1_ref, g1_ref, degp_ref, b1_ref, w2_ref, g2_ref):
  dis = _dis_from_degp(degp_ref[...])
  s = s1_ref[0] + s1_ref[1]
  out1 = dis * (s + g1_ref[...]) + b1_ref[...]
  u = _SELU_SCALE * jnp.where(
      out1 > 0, out1, _SELU_ALPHA * (jnp.exp(out1) - 1.0))
  m = jnp.dot(u, w2_ref[...], preferred_element_type=jnp.float32)
  g2_ref[...] = m * dis


def _tc_stage3(s2_ref, g2_ref, degp_ref, b2_ref, out_ref):
  dis = _dis_from_degp(degp_ref[...])
  s = s2_ref[0] + s2_ref[1]
  out_ref[...] = dis * (s + g2_ref[...]) + b2_ref[...]


def _row_spec(d):
  return pl.BlockSpec((_ROW_BLK, d), lambda i: (i, 0))


def _part_spec(d):
  return pl.BlockSpec((NC, _ROW_BLK, d), lambda i: (0, i, 0))


_DEGP_SPEC = pl.BlockSpec((NC, _ROW_BLK, DEG_W), lambda i: (0, i, 0))


def _full_spec(shape):
  nd = len(shape)
  return pl.BlockSpec(shape, lambda i: (0,) * nd)


def kernel(x, edge_index, W1, b1, W2, b2):
  src = edge_index[0].astype(jnp.int32).reshape(NW, NCHUNK, CH)
  dst = edge_index[1].astype(jnp.int32).reshape(NW, NCHUNK, CH)

  ones_ch = jnp.ones((CH, DEG_W), jnp.float32)
  zeros1 = jnp.zeros((N, DEG_W), jnp.float32)
  zeros_hid = jnp.zeros((N, HID_DIM), jnp.float32)
  zeros_out = jnp.zeros((N, OUT_DIM), jnp.float32)

  degp = _make_sc_degree()(ones_ch, dst, zeros1)

  g1 = pl.pallas_call(
      _tc_stage1,
      grid=(_GRID,),
      in_specs=[_row_spec(IN_DIM), _full_spec((IN_DIM, HID_DIM)), _DEGP_SPEC],
      out_specs=_row_spec(HID_DIM),
      out_shape=jax.ShapeDtypeStruct((N, HID_DIM), jnp.float32),
  )(x, W1, degp)

  s1 = _make_sc_scatter(HID_DIM)(g1, src, dst, zeros_hid)

  g2 = pl.pallas_call(
      _tc_stage2,
      grid=(_GRID,),
      in_specs=[_part_spec(HID_DIM), _row_spec(HID_DIM), _DEGP_SPEC,
                _full_spec((1, HID_DIM)), _full_spec((HID_DIM, OUT_DIM))],
      out_specs=_row_spec(OUT_DIM),
      out_shape=jax.ShapeDtypeStruct((N, OUT_DIM), jnp.float32),
  )(s1, g1, degp, b1.reshape(1, HID_DIM), W2)

  s2 = _make_sc_scatter(OUT_DIM)(g2, src, dst, zeros_out)

  out = pl.pallas_call(
      _tc_stage3,
      grid=(_GRID,),
      in_specs=[_part_spec(OUT_DIM), _row_spec(OUT_DIM), _DEGP_SPEC,
                _full_spec((1, OUT_DIM))],
      out_specs=_row_spec(OUT_DIM),
      out_shape=jax.ShapeDtypeStruct((N, OUT_DIM), jnp.float32),
  )(s2, g2, degp, b2.reshape(1, OUT_DIM))

  return out


# speedup vs baseline: 25.2817x; 1.0461x over previous
import functools

import jax
import jax.numpy as jnp
from jax import lax
from jax.experimental import pallas as pl
from jax.experimental.pallas import tpu as pltpu
from jax.experimental.pallas import tpu_sc as plsc

N = 10000
E = 320000
IN_DIM = 128
HID_DIM = 128
OUT_DIM = 64
HALF = HID_DIM // 2

NC = 2
NS = 16
NW = NC * NS
CH = 125
EPW = E // NW
NCHUNK = EPW // CH
EPS = E // NS
NCHUNK1 = EPS // CH
ROWS_PT = N // NS

_SELU_ALPHA = 1.6732632423543772
_SELU_SCALE = 1.0507009873554805


def _make_sc_scatter_l1():
  mesh = plsc.VectorSubcoreMesh(core_axis_name="c", subcore_axis_name="s")

  @functools.partial(
      pl.kernel,
      out_type=jax.ShapeDtypeStruct((NC, N, HALF), jnp.float32),
      mesh=mesh,
      compiler_params=pltpu.CompilerParams(use_tc_tiling_on_sc=False),
      scratch_types=[
          pltpu.VMEM((NCHUNK1, CH), jnp.int32),
          pltpu.VMEM((NCHUNK1, CH), jnp.int32),
          pltpu.VMEM((2, CH, HALF), jnp.float32),
          pltpu.VMEM_SHARED((N, HALF), jnp.float32),
          pltpu.SemaphoreType.DMA,
      ],
  )
  def sc_scatter1(table_hbm, src_hbm, dst_hbm, zeros_hbm, out_hbm,
                  src_v, dst_v, buf, acc, sem):
    cid = lax.axis_index("c")
    sid = lax.axis_index("s")

    pltpu.sync_copy(zeros_hbm.at[pl.ds(sid * ROWS_PT, ROWS_PT)],
                    acc.at[pl.ds(sid * ROWS_PT, ROWS_PT)])
    pltpu.sync_copy(src_hbm.at[sid], src_v)
    pltpu.sync_copy(dst_hbm.at[sid], dst_v)
    plsc.subcore_barrier()

    pltpu.async_copy(table_hbm.at[cid].at[src_v.at[0]], buf.at[0], sem)

    def body(j, carry):
      slot = lax.rem(j, 2)
      pltpu.make_async_copy(table_hbm.at[cid].at[src_v.at[j]], buf.at[slot],
                            sem).wait()

      @pl.when(j + 1 < NCHUNK1)
      def _():
        pltpu.async_copy(table_hbm.at[cid].at[src_v.at[j + 1]],
                         buf.at[1 - slot], sem)

      pltpu.sync_copy(buf.at[slot], acc.at[dst_v.at[j]], add=True)
      return carry

    lax.fori_loop(0, NCHUNK1, body, 0, unroll=False)
    plsc.subcore_barrier()
    pltpu.sync_copy(acc.at[pl.ds(sid * ROWS_PT, ROWS_PT)],
                    out_hbm.at[cid, pl.ds(sid * ROWS_PT, ROWS_PT)])

  return sc_scatter1


def _make_sc_scatter_l2():
  mesh = plsc.VectorSubcoreMesh(core_axis_name="c", subcore_axis_name="s")

  @functools.partial(
      pl.kernel,
      out_type=jax.ShapeDtypeStruct((NC, N, OUT_DIM), jnp.float32),
      mesh=mesh,
      compiler_params=pltpu.CompilerParams(use_tc_tiling_on_sc=False),
      scratch_types=[
          pltpu.VMEM((NCHUNK, CH), jnp.int32),
          pltpu.VMEM((NCHUNK, CH), jnp.int32),
          pltpu.VMEM((2, CH, OUT_DIM), jnp.float32),
          pltpu.VMEM_SHARED((N, OUT_DIM), jnp.float32),
          pltpu.SemaphoreType.DMA,
      ],
  )
  def sc_scatter2(table_hbm, src_hbm, dst_hbm, zeros_hbm, out_hbm,
                  src_v, dst_v, buf, acc, sem):
    cid = lax.axis_index("c")
    sid = lax.axis_index("s")
    wid = sid * NC + cid

    pltpu.sync_copy(zeros_hbm.at[pl.ds(sid * ROWS_PT, ROWS_PT)],
                    acc.at[pl.ds(sid * ROWS_PT, ROWS_PT)])
    pltpu.sync_copy(src_hbm.at[wid], src_v)
    pltpu.sync_copy(dst_hbm.at[wid], dst_v)
    plsc.subcore_barrier()

    pltpu.async_copy(table_hbm.at[src_v.at[0]], buf.at[0], sem)

    def body(j, carry):
      slot = lax.rem(j, 2)
      pltpu.make_async_copy(table_hbm.at[src_v.at[j]], buf.at[slot],
                            sem).wait()

      @pl.when(j + 1 < NCHUNK)
      def _():
        pltpu.async_copy(table_hbm.at[src_v.at[j + 1]], buf.at[1 - slot],
                         sem)

      pltpu.sync_copy(buf.at[slot], acc.at[dst_v.at[j]], add=True)
      return carry

    lax.fori_loop(0, NCHUNK, body, 0, unroll=False)
    plsc.subcore_barrier()
    pltpu.sync_copy(acc.at[pl.ds(sid * ROWS_PT, ROWS_PT)],
                    out_hbm.at[cid, pl.ds(sid * ROWS_PT, ROWS_PT)])

  return sc_scatter2


DEG_W = 16


def _make_sc_degree():
  mesh = plsc.VectorSubcoreMesh(core_axis_name="c", subcore_axis_name="s")

  @functools.partial(
      pl.kernel,
      out_type=jax.ShapeDtypeStruct((NC, N, DEG_W), jnp.float32),
      mesh=mesh,
      compiler_params=pltpu.CompilerParams(use_tc_tiling_on_sc=False),
      scratch_types=[
          pltpu.VMEM((NCHUNK, CH), jnp.int32),
          pltpu.VMEM((CH, DEG_W), jnp.float32),
          pltpu.VMEM_SHARED((N, DEG_W), jnp.float32),
          pltpu.SemaphoreType.DMA,
      ],
  )
  def sc_degree(ones_hbm, dst_hbm, zeros_hbm, out_hbm, dst_v, buf, acc, sem):
    cid = lax.axis_index("c")
    sid = lax.axis_index("s")
    wid = sid * NC + cid

    @pl.when(sid == 0)
    def _():
      pltpu.sync_copy(zeros_hbm, acc)

    pltpu.sync_copy(dst_hbm.at[wid], dst_v)
    pltpu.sync_copy(ones_hbm, buf)
    plsc.subcore_barrier()

    def body(j, carry):
      pltpu.sync_copy(buf, acc.at[dst_v.at[j]], add=True)
      return carry

    lax.fori_loop(0, NCHUNK, body, 0, unroll=False)
    plsc.subcore_barrier()

    @pl.when(sid == 0)
    def _():
      pltpu.sync_copy(acc, out_hbm.at[cid])

  return sc_degree


_ROW_BLK = 1000
_GRID = N // _ROW_BLK


def _dis_from_degp(degp_blk):
  deg = degp_blk[0, :, 0:1] + degp_blk[1, :, 0:1] + 1.0
  return lax.rsqrt(deg)


def _tc_stage1(x_ref, w1_ref, degp_ref, g1_ref):
  dis = _dis_from_degp(degp_ref[...])
  h = jnp.dot(x_ref[...], w1_ref[...], preferred_element_type=jnp.float32)
  g = h * dis
  g1_ref[0] = g[:, :HALF]
  g1_ref[1] = g[:, HALF:]


def _tc_stage2(s1_ref, g1_ref, degp_ref, b1_ref, w2_ref, g2_ref):
  dis = _dis_from_degp(degp_ref[...])
  t0 = dis * (s1_ref[0] + g1_ref[0]) + b1_ref[0]
  t1 = dis * (s1_ref[1] + g1_ref[1]) + b1_ref[1]
  out1 = jnp.concatenate([t0, t1], axis=1)
  u = _SELU_SCALE * jnp.where(
      out1 > 0, out1, _SELU_ALPHA * (jnp.exp(out1) - 1.0))
  m = jnp.dot(u, w2_ref[...], preferred_element_type=jnp.float32)
  g2_ref[...] = m * dis


def _tc_stage3(s2_ref, g2_ref, degp_ref, b2_ref, out_ref):
  dis = _dis_from_degp(degp_ref[...])
  s = s2_ref[0] + s2_ref[1]
  out_ref[...] = dis * (s + g2_ref[...]) + b2_ref[...]


def _row_spec(d):
  return pl.BlockSpec((_ROW_BLK, d), lambda i: (i, 0))


def _half_spec():
  return pl.BlockSpec((NC, _ROW_BLK, HALF), lambda i: (0, i, 0))


_DEGP_SPEC = pl.BlockSpec((NC, _ROW_BLK, DEG_W), lambda i: (0, i, 0))


def _full_spec(shape):
  nd = len(shape)
  return pl.BlockSpec(shape, lambda i: (0,) * nd)


def kernel(x, edge_index, W1, b1, W2, b2):
  src32 = edge_index[0].astype(jnp.int32)
  dst32 = edge_index[1].astype(jnp.int32)
  src_w = src32.reshape(NW, NCHUNK, CH)
  dst_w = dst32.reshape(NW, NCHUNK, CH)
  src_s = src32.reshape(NS, NCHUNK1, CH)
  dst_s = dst32.reshape(NS, NCHUNK1, CH)

  ones_ch = jnp.ones((CH, DEG_W), jnp.float32)
  zeros_deg = jnp.zeros((N, DEG_W), jnp.float32)
  zeros_half = jnp.zeros((N, HALF), jnp.float32)
  zeros_out = jnp.zeros((N, OUT_DIM), jnp.float32)

  degp = _make_sc_degree()(ones_ch, dst_w, zeros_deg)

  g1 = pl.pallas_call(
      _tc_stage1,
      grid=(_GRID,),
      in_specs=[_row_spec(IN_DIM), _full_spec((IN_DIM, HID_DIM)), _DEGP_SPEC],
      out_specs=_half_spec(),
      out_shape=jax.ShapeDtypeStruct((NC, N, HALF), jnp.float32),
  )(x, W1, degp)

  s1 = _make_sc_scatter_l1()(g1, src_s, dst_s, zeros_half)

  g2 = pl.pallas_call(
      _tc_stage2,
      grid=(_GRID,),
      in_specs=[_half_spec(), _half_spec(), _DEGP_SPEC,
                _full_spec((NC, 1, HALF)), _full_spec((HID_DIM, OUT_DIM))],
      out_specs=_row_spec(OUT_DIM),
      out_shape=jax.ShapeDtypeStruct((N, OUT_DIM), jnp.float32),
  )(s1, g1, degp, b1.reshape(NC, 1, HALF), W2)

  s2 = _make_sc_scatter_l2()(g2, src_w, dst_w, zeros_out)

  out = pl.pallas_call(
      _tc_stage3,
      grid=(_GRID,),
      in_specs=[pl.BlockSpec((NC, _ROW_BLK, OUT_DIM), lambda i: (0, i, 0)),
                _row_spec(OUT_DIM), _DEGP_SPEC, _full_spec((1, OUT_DIM))],
      out_specs=_row_spec(OUT_DIM),
      out_shape=jax.ShapeDtypeStruct((N, OUT_DIM), jnp.float32),
  )(s2, g2, degp, b2.reshape(1, OUT_DIM))

  return out


# speedup vs baseline: 35.7259x; 1.4131x over previous
import functools

import jax
import jax.numpy as jnp
from jax import lax
from jax.experimental import pallas as pl
from jax.experimental.pallas import tpu as pltpu
from jax.experimental.pallas import tpu_sc as plsc

N = 10000
E = 320000
IN_DIM = 128
HID_DIM = 128
OUT_DIM = 64
HALF = HID_DIM // 2

NC = 2
NS = 16
NW = NC * NS
CH = 125
EPW = E // NW
NCHUNK = EPW // CH
EPS = E // NS
NCHUNK1 = EPS // CH
ROWS_PT = N // NS
NBUF = 4

_SELU_ALPHA = 1.6732632423543772
_SELU_SCALE = 1.0507009873554805


def _make_sc_scatter_l1():
  mesh = plsc.VectorSubcoreMesh(core_axis_name="c", subcore_axis_name="s")

  @functools.partial(
      pl.kernel,
      out_type=jax.ShapeDtypeStruct((NC, N, HALF), jnp.float32),
      mesh=mesh,
      compiler_params=pltpu.CompilerParams(use_tc_tiling_on_sc=False),
      scratch_types=[
          pltpu.VMEM((NCHUNK1, CH), jnp.int32),
          pltpu.VMEM((NCHUNK1, CH), jnp.int32),
          pltpu.VMEM((NBUF, CH, HALF), jnp.float32),
          pltpu.VMEM_SHARED((N, HALF), jnp.float32),
          pltpu.SemaphoreType.DMA,
          pltpu.SemaphoreType.DMA,
      ],
  )
  def sc_scatter1(table_hbm, src_hbm, dst_hbm, zeros_hbm, out_hbm,
                  src_v, dst_v, buf, acc, gsem, ssem):
    cid = lax.axis_index("c")
    sid = lax.axis_index("s")

    pltpu.sync_copy(zeros_hbm.at[pl.ds(sid * ROWS_PT, ROWS_PT)],
                    acc.at[pl.ds(sid * ROWS_PT, ROWS_PT)])
    pltpu.sync_copy(src_hbm.at[sid], src_v)
    pltpu.sync_copy(dst_hbm.at[sid], dst_v)
    plsc.subcore_barrier()

    def gather(j):
      return pltpu.make_async_copy(
          table_hbm.at[cid].at[src_v.at[j]], buf.at[lax.rem(j, NBUF)], gsem)

    def scat(j):
      return pltpu.make_async_copy(
          buf.at[lax.rem(j, NBUF)], acc.at[dst_v.at[j]], ssem)

    for b in range(NBUF - 1):
      gather(b).start()

    def body(j, carry):
      gather(j).wait()
      scat(j).start(add=True)

      @pl.when(j >= 1)
      def _():
        scat(j - 1).wait()

      @pl.when(j + NBUF - 1 < NCHUNK1)
      def _():
        gather(j + NBUF - 1).start()

      return carry

    lax.fori_loop(0, NCHUNK1, body, 0, unroll=False)
    scat(NCHUNK1 - 1).wait()
    plsc.subcore_barrier()
    pltpu.sync_copy(acc.at[pl.ds(sid * ROWS_PT, ROWS_PT)],
                    out_hbm.at[cid, pl.ds(sid * ROWS_PT, ROWS_PT)])

  return sc_scatter1


def _make_sc_scatter_l2():
  mesh = plsc.VectorSubcoreMesh(core_axis_name="c", subcore_axis_name="s")

  @functools.partial(
      pl.kernel,
      out_type=jax.ShapeDtypeStruct((NC, N, OUT_DIM), jnp.float32),
      mesh=mesh,
      compiler_params=pltpu.CompilerParams(use_tc_tiling_on_sc=False),
      scratch_types=[
          pltpu.VMEM((NCHUNK, CH), jnp.int32),
          pltpu.VMEM((NCHUNK, CH), jnp.int32),
          pltpu.VMEM((NBUF, CH, OUT_DIM), jnp.float32),
          pltpu.VMEM_SHARED((N, OUT_DIM), jnp.float32),
          pltpu.SemaphoreType.DMA,
          pltpu.SemaphoreType.DMA,
      ],
  )
  def sc_scatter2(table_hbm, src_hbm, dst_hbm, zeros_hbm, out_hbm,
                  src_v, dst_v, buf, acc, gsem, ssem):
    cid = lax.axis_index("c")
    sid = lax.axis_index("s")
    wid = sid * NC + cid

    pltpu.sync_copy(zeros_hbm.at[pl.ds(sid * ROWS_PT, ROWS_PT)],
                    acc.at[pl.ds(sid * ROWS_PT, ROWS_PT)])
    pltpu.sync_copy(src_hbm.at[wid], src_v)
    pltpu.sync_copy(dst_hbm.at[wid], dst_v)
    plsc.subcore_barrier()

    def gather(j):
      return pltpu.make_async_copy(
          table_hbm.at[src_v.at[j]], buf.at[lax.rem(j, NBUF)], gsem)

    def scat(j):
      return pltpu.make_async_copy(
          buf.at[lax.rem(j, NBUF)], acc.at[dst_v.at[j]], ssem)

    for b in range(NBUF - 1):
      gather(b).start()

    def body(j, carry):
      gather(j).wait()
      scat(j).start(add=True)

      @pl.when(j >= 1)
      def _():
        scat(j - 1).wait()

      @pl.when(j + NBUF - 1 < NCHUNK)
      def _():
        gather(j + NBUF - 1).start()

      return carry

    lax.fori_loop(0, NCHUNK, body, 0, unroll=False)
    scat(NCHUNK - 1).wait()
    plsc.subcore_barrier()
    pltpu.sync_copy(acc.at[pl.ds(sid * ROWS_PT, ROWS_PT)],
                    out_hbm.at[cid, pl.ds(sid * ROWS_PT, ROWS_PT)])

  return sc_scatter2


DEG_W = 16


def _make_sc_degree():
  mesh = plsc.VectorSubcoreMesh(core_axis_name="c", subcore_axis_name="s")

  @functools.partial(
      pl.kernel,
      out_type=jax.ShapeDtypeStruct((NC, N, DEG_W), jnp.float32),
      mesh=mesh,
      compiler_params=pltpu.CompilerParams(use_tc_tiling_on_sc=False),
      scratch_types=[
          pltpu.VMEM((NCHUNK, CH), jnp.int32),
          pltpu.VMEM((CH, DEG_W), jnp.float32),
          pltpu.VMEM_SHARED((N, DEG_W), jnp.float32),
          pltpu.SemaphoreType.DMA,
      ],
  )
  def sc_degree(ones_hbm, dst_hbm, zeros_hbm, out_hbm, dst_v, buf, acc, sem):
    cid = lax.axis_index("c")
    sid = lax.axis_index("s")
    wid = sid * NC + cid

    @pl.when(sid == 0)
    def _():
      pltpu.sync_copy(zeros_hbm, acc)

    pltpu.sync_copy(dst_hbm.at[wid], dst_v)
    pltpu.sync_copy(ones_hbm, buf)
    plsc.subcore_barrier()

    def body(j, carry):
      pltpu.sync_copy(buf, acc.at[dst_v.at[j]], add=True)
      return carry

    lax.fori_loop(0, NCHUNK, body, 0, unroll=False)
    plsc.subcore_barrier()

    @pl.when(sid == 0)
    def _():
      pltpu.sync_copy(acc, out_hbm.at[cid])

  return sc_degree


_ROW_BLK = 1000
_GRID = N // _ROW_BLK


def _dis_from_degp(degp_blk):
  deg = degp_blk[0, :, 0:1] + degp_blk[1, :, 0:1] + 1.0
  return lax.rsqrt(deg)


def _tc_stage1(x_ref, w1_ref, degp_ref, g1_ref):
  dis = _dis_from_degp(degp_ref[...])
  h = jnp.dot(x_ref[...], w1_ref[...], preferred_element_type=jnp.float32)
  g = h * dis
  g1_ref[0] = g[:, :HALF]
  g1_ref[1] = g[:, HALF:]


def _tc_stage2(s1_ref, g1_ref, degp_ref, b1_ref, w2_ref, g2_ref):
  dis = _dis_from_degp(degp_ref[...])
  t0 = dis * (s1_ref[0] + g1_ref[0]) + b1_ref[0]
  t1 = dis * (s1_ref[1] + g1_ref[1]) + b1_ref[1]
  out1 = jnp.concatenate([t0, t1], axis=1)
  u = _SELU_SCALE * jnp.where(
      out1 > 0, out1, _SELU_ALPHA * (jnp.exp(out1) - 1.0))
  m = jnp.dot(u, w2_ref[...], preferred_element_type=jnp.float32)
  g2_ref[...] = m * dis


def _tc_stage3(s2_ref, g2_ref, degp_ref, b2_ref, out_ref):
  dis = _dis_from_degp(degp_ref[...])
  s = s2_ref[0] + s2_ref[1]
  out_ref[...] = dis * (s + g2_ref[...]) + b2_ref[...]


def _row_spec(d):
  return pl.BlockSpec((_ROW_BLK, d), lambda i: (i, 0))


def _half_spec():
  return pl.BlockSpec((NC, _ROW_BLK, HALF), lambda i: (0, i, 0))


_DEGP_SPEC = pl.BlockSpec((NC, _ROW_BLK, DEG_W), lambda i: (0, i, 0))


def _full_spec(shape):
  nd = len(shape)
  return pl.BlockSpec(shape, lambda i: (0,) * nd)


def kernel(x, edge_index, W1, b1, W2, b2):
  src32 = edge_index[0].astype(jnp.int32)
  dst32 = edge_index[1].astype(jnp.int32)
  src_w = src32.reshape(NW, NCHUNK, CH)
  dst_w = dst32.reshape(NW, NCHUNK, CH)
  src_s = src32.reshape(NS, NCHUNK1, CH)
  dst_s = dst32.reshape(NS, NCHUNK1, CH)

  ones_ch = jnp.ones((CH, DEG_W), jnp.float32)
  zeros_deg = jnp.zeros((N, DEG_W), jnp.float32)
  zeros_half = jnp.zeros((N, HALF), jnp.float32)
  zeros_out = jnp.zeros((N, OUT_DIM), jnp.float32)

  degp = _make_sc_degree()(ones_ch, dst_w, zeros_deg)

  g1 = pl.pallas_call(
      _tc_stage1,
      grid=(_GRID,),
      in_specs=[_row_spec(IN_DIM), _full_spec((IN_DIM, HID_DIM)), _DEGP_SPEC],
      out_specs=_half_spec(),
      out_shape=jax.ShapeDtypeStruct((NC, N, HALF), jnp.float32),
  )(x, W1, degp)

  s1 = _make_sc_scatter_l1()(g1, src_s, dst_s, zeros_half)

  g2 = pl.pallas_call(
      _tc_stage2,
      grid=(_GRID,),
      in_specs=[_half_spec(), _half_spec(), _DEGP_SPEC,
                _full_spec((NC, 1, HALF)), _full_spec((HID_DIM, OUT_DIM))],
      out_specs=_row_spec(OUT_DIM),
      out_shape=jax.ShapeDtypeStruct((N, OUT_DIM), jnp.float32),
  )(s1, g1, degp, b1.reshape(NC, 1, HALF), W2)

  s2 = _make_sc_scatter_l2()(g2, src_w, dst_w, zeros_out)

  out = pl.pallas_call(
      _tc_stage3,
      grid=(_GRID,),
      in_specs=[pl.BlockSpec((NC, _ROW_BLK, OUT_DIM), lambda i: (0, i, 0)),
                _row_spec(OUT_DIM), _DEGP_SPEC, _full_spec((1, OUT_DIM))],
      out_specs=_row_spec(OUT_DIM),
      out_shape=jax.ShapeDtypeStruct((N, OUT_DIM), jnp.float32),
  )(s2, g2, degp, b2.reshape(1, OUT_DIM))

  return out


# speedup vs baseline: 36.3493x; 1.0174x over previous
import functools

import jax
import jax.numpy as jnp
from jax import lax
from jax.experimental import pallas as pl
from jax.experimental.pallas import tpu as pltpu
from jax.experimental.pallas import tpu_sc as plsc

N = 10000
E = 320000
IN_DIM = 128
HID_DIM = 128
OUT_DIM = 64
HALF = HID_DIM // 2

NC = 2
NS = 16
NW = NC * NS
CH = 125
EPW = E // NW
NCHUNK = EPW // CH
EPS = E // NS
NCHUNK1 = EPS // CH
ROWS_PT = N // NS
NBUF = 4

_SELU_ALPHA = 1.6732632423543772
_SELU_SCALE = 1.0507009873554805


def _make_sc_scatter_l1():
  mesh = plsc.VectorSubcoreMesh(core_axis_name="c", subcore_axis_name="s")

  @functools.partial(
      pl.kernel,
      out_type=jax.ShapeDtypeStruct((NC, N, HALF), jnp.float32),
      mesh=mesh,
      compiler_params=pltpu.CompilerParams(use_tc_tiling_on_sc=False,
                                           skip_device_barrier=True),
      scratch_types=[
          pltpu.VMEM((NCHUNK1, CH), jnp.int32),
          pltpu.VMEM((NCHUNK1, CH), jnp.int32),
          pltpu.VMEM((NBUF, CH, HALF), jnp.float32),
          pltpu.VMEM_SHARED((N, HALF), jnp.float32),
          pltpu.SemaphoreType.DMA,
          pltpu.SemaphoreType.DMA,
      ],
  )
  def sc_scatter1(table_hbm, src_hbm, dst_hbm, zeros_hbm, out_hbm,
                  src_v, dst_v, buf, acc, gsem, ssem):
    cid = lax.axis_index("c")
    sid = lax.axis_index("s")

    pltpu.sync_copy(zeros_hbm.at[pl.ds(sid * ROWS_PT, ROWS_PT)],
                    acc.at[pl.ds(sid * ROWS_PT, ROWS_PT)])
    pltpu.sync_copy(src_hbm.at[sid], src_v)
    pltpu.sync_copy(dst_hbm.at[sid], dst_v)
    plsc.subcore_barrier()

    def gather(j):
      return pltpu.make_async_copy(
          table_hbm.at[cid].at[src_v.at[j]], buf.at[lax.rem(j, NBUF)], gsem)

    def scat(j):
      return pltpu.make_async_copy(
          buf.at[lax.rem(j, NBUF)], acc.at[dst_v.at[j]], ssem)

    for b in range(NBUF - 1):
      gather(b).start()

    def body(j, carry):
      gather(j).wait()
      scat(j).start(add=True)

      @pl.when(j >= 1)
      def _():
        scat(j - 1).wait()

      @pl.when(j + NBUF - 1 < NCHUNK1)
      def _():
        gather(j + NBUF - 1).start()

      return carry

    lax.fori_loop(0, NCHUNK1, body, 0, unroll=False)
    scat(NCHUNK1 - 1).wait()
    plsc.subcore_barrier()
    pltpu.sync_copy(acc.at[pl.ds(sid * ROWS_PT, ROWS_PT)],
                    out_hbm.at[cid, pl.ds(sid * ROWS_PT, ROWS_PT)])

  return sc_scatter1


def _make_sc_scatter_l2():
  mesh = plsc.VectorSubcoreMesh(core_axis_name="c", subcore_axis_name="s")

  @functools.partial(
      pl.kernel,
      out_type=jax.ShapeDtypeStruct((NC, N, OUT_DIM), jnp.float32),
      mesh=mesh,
      compiler_params=pltpu.CompilerParams(use_tc_tiling_on_sc=False,
                                           skip_device_barrier=True),
      scratch_types=[
          pltpu.VMEM((NCHUNK, CH), jnp.int32),
          pltpu.VMEM((NCHUNK, CH), jnp.int32),
          pltpu.VMEM((NBUF, CH, OUT_DIM), jnp.float32),
          pltpu.VMEM_SHARED((N, OUT_DIM), jnp.float32),
          pltpu.SemaphoreType.DMA,
          pltpu.SemaphoreType.DMA,
      ],
  )
  def sc_scatter2(table_hbm, src_hbm, dst_hbm, zeros_hbm, out_hbm,
                  src_v, dst_v, buf, acc, gsem, ssem):
    cid = lax.axis_index("c")
    sid = lax.axis_index("s")
    wid = sid * NC + cid

    pltpu.sync_copy(zeros_hbm.at[pl.ds(sid * ROWS_PT, ROWS_PT)],
                    acc.at[pl.ds(sid * ROWS_PT, ROWS_PT)])
    pltpu.sync_copy(src_hbm.at[wid], src_v)
    pltpu.sync_copy(dst_hbm.at[wid], dst_v)
    plsc.subcore_barrier()

    def gather(j):
      return pltpu.make_async_copy(
          table_hbm.at[src_v.at[j]], buf.at[lax.rem(j, NBUF)], gsem)

    def scat(j):
      return pltpu.make_async_copy(
          buf.at[lax.rem(j, NBUF)], acc.at[dst_v.at[j]], ssem)

    for b in range(NBUF - 1):
      gather(b).start()

    def body(j, carry):
      gather(j).wait()
      scat(j).start(add=True)

      @pl.when(j >= 1)
      def _():
        scat(j - 1).wait()

      @pl.when(j + NBUF - 1 < NCHUNK)
      def _():
        gather(j + NBUF - 1).start()

      return carry

    lax.fori_loop(0, NCHUNK, body, 0, unroll=False)
    scat(NCHUNK - 1).wait()
    plsc.subcore_barrier()
    pltpu.sync_copy(acc.at[pl.ds(sid * ROWS_PT, ROWS_PT)],
                    out_hbm.at[cid, pl.ds(sid * ROWS_PT, ROWS_PT)])

  return sc_scatter2


DEG_W = 16


def _make_sc_degree():
  mesh = plsc.VectorSubcoreMesh(core_axis_name="c", subcore_axis_name="s")

  @functools.partial(
      pl.kernel,
      out_type=jax.ShapeDtypeStruct((NC, N, DEG_W), jnp.float32),
      mesh=mesh,
      compiler_params=pltpu.CompilerParams(use_tc_tiling_on_sc=False,
                                           skip_device_barrier=True),
      scratch_types=[
          pltpu.VMEM((NCHUNK, CH), jnp.int32),
          pltpu.VMEM((CH, DEG_W), jnp.float32),
          pltpu.VMEM_SHARED((N, DEG_W), jnp.float32),
          pltpu.SemaphoreType.DMA,
      ],
  )
  def sc_degree(ones_hbm, dst_hbm, zeros_hbm, out_hbm, dst_v, buf, acc, sem):
    cid = lax.axis_index("c")
    sid = lax.axis_index("s")
    wid = sid * NC + cid

    @pl.when(sid == 0)
    def _():
      pltpu.sync_copy(zeros_hbm, acc)

    pltpu.sync_copy(dst_hbm.at[wid], dst_v)
    pltpu.sync_copy(ones_hbm, buf)
    plsc.subcore_barrier()

    def body(j, carry):
      pltpu.sync_copy(buf, acc.at[dst_v.at[j]], add=True)
      return carry

    lax.fori_loop(0, NCHUNK, body, 0, unroll=False)
    plsc.subcore_barrier()

    @pl.when(sid == 0)
    def _():
      pltpu.sync_copy(acc, out_hbm.at[cid])

  return sc_degree


_ROW_BLK = 2000
_GRID = N // _ROW_BLK


def _dis_from_degp(degp_blk):
  deg = degp_blk[0, :, 0:1] + degp_blk[1, :, 0:1] + 1.0
  return lax.rsqrt(deg)


def _tc_stage1(x_ref, w1_ref, degp_ref, g1_ref):
  dis = _dis_from_degp(degp_ref[...])
  h = jnp.dot(x_ref[...], w1_ref[...], preferred_element_type=jnp.float32)
  g = h * dis
  g1_ref[0] = g[:, :HALF]
  g1_ref[1] = g[:, HALF:]


def _tc_stage2(s1_ref, g1_ref, degp_ref, b1_ref, w2_ref, g2_ref):
  dis = _dis_from_degp(degp_ref[...])
  t0 = dis * (s1_ref[0] + g1_ref[0]) + b1_ref[0]
  t1 = dis * (s1_ref[1] + g1_ref[1]) + b1_ref[1]
  out1 = jnp.concatenate([t0, t1], axis=1)
  u = _SELU_SCALE * jnp.where(
      out1 > 0, out1, _SELU_ALPHA * (jnp.exp(out1) - 1.0))
  m = jnp.dot(u, w2_ref[...], preferred_element_type=jnp.float32)
  g2_ref[...] = m * dis


def _tc_stage3(s2_ref, g2_ref, degp_ref, b2_ref, out_ref):
  dis = _dis_from_degp(degp_ref[...])
  s = s2_ref[0] + s2_ref[1]
  out_ref[...] = dis * (s + g2_ref[...]) + b2_ref[...]


def _row_spec(d):
  return pl.BlockSpec((_ROW_BLK, d), lambda i: (i, 0))


def _half_spec():
  return pl.BlockSpec((NC, _ROW_BLK, HALF), lambda i: (0, i, 0))


_DEGP_SPEC = pl.BlockSpec((NC, _ROW_BLK, DEG_W), lambda i: (0, i, 0))


def _full_spec(shape):
  nd = len(shape)
  return pl.BlockSpec(shape, lambda i: (0,) * nd)


def kernel(x, edge_index, W1, b1, W2, b2):
  src32 = edge_index[0].astype(jnp.int32)
  dst32 = edge_index[1].astype(jnp.int32)
  src_w = src32.reshape(NW, NCHUNK, CH)
  dst_w = dst32.reshape(NW, NCHUNK, CH)
  src_s = src32.reshape(NS, NCHUNK1, CH)
  dst_s = dst32.reshape(NS, NCHUNK1, CH)

  ones_ch = jnp.ones((CH, DEG_W), jnp.float32)
  zeros_deg = jnp.zeros((N, DEG_W), jnp.float32)
  zeros_half = jnp.zeros((N, HALF), jnp.float32)
  zeros_out = jnp.zeros((N, OUT_DIM), jnp.float32)

  degp = _make_sc_degree()(ones_ch, dst_w, zeros_deg)

  _tc_params = pltpu.CompilerParams(skip_device_barrier=True)
  g1 = pl.pallas_call(
      _tc_stage1,
      grid=(_GRID,),
      compiler_params=_tc_params,
      in_specs=[_row_spec(IN_DIM), _full_spec((IN_DIM, HID_DIM)), _DEGP_SPEC],
      out_specs=_half_spec(),
      out_shape=jax.ShapeDtypeStruct((NC, N, HALF), jnp.float32),
  )(x, W1, degp)

  s1 = _make_sc_scatter_l1()(g1, src_s, dst_s, zeros_half)

  g2 = pl.pallas_call(
      _tc_stage2,
      grid=(_GRID,),
      compiler_params=_tc_params,
      in_specs=[_half_spec(), _half_spec(), _DEGP_SPEC,
                _full_spec((NC, 1, HALF)), _full_spec((HID_DIM, OUT_DIM))],
      out_specs=_row_spec(OUT_DIM),
      out_shape=jax.ShapeDtypeStruct((N, OUT_DIM), jnp.float32),
  )(s1, g1, degp, b1.reshape(NC, 1, HALF), W2)

  s2 = _make_sc_scatter_l2()(g2, src_w, dst_w, zeros_out)

  out = pl.pallas_call(
      _tc_stage3,
      grid=(_GRID,),
      compiler_params=_tc_params,
      in_specs=[pl.BlockSpec((NC, _ROW_BLK, OUT_DIM), lambda i: (0, i, 0)),
                _row_spec(OUT_DIM), _DEGP_SPEC, _full_spec((1, OUT_DIM))],
      out_specs=_row_spec(OUT_DIM),
      out_shape=jax.ShapeDtypeStruct((N, OUT_DIM), jnp.float32),
  )(s2, g2, degp, b2.reshape(1, OUT_DIM))

  return out


# speedup vs baseline: 36.4095x; 1.0017x over previous
import functools

import jax
import jax.numpy as jnp
from jax import lax
from jax.experimental import pallas as pl
from jax.experimental.pallas import tpu as pltpu
from jax.experimental.pallas import tpu_sc as plsc

N = 10000
E = 320000
IN_DIM = 128
HID_DIM = 128
OUT_DIM = 64
HALF = HID_DIM // 2

NC = 2
NS = 16
NW = NC * NS
CH = 125
EPW = E // NW
NCHUNK = EPW // CH
EPS = E // NS
NCHUNK1 = EPS // CH
ROWS_PT = N // NS
NBUF = 4

_SELU_ALPHA = 1.6732632423543772
_SELU_SCALE = 1.0507009873554805


def _make_sc_scatter_l1():
  mesh = plsc.VectorSubcoreMesh(core_axis_name="c", subcore_axis_name="s")

  @functools.partial(
      pl.kernel,
      out_type=jax.ShapeDtypeStruct((NC, N, HALF), jnp.float32),
      mesh=mesh,
      compiler_params=pltpu.CompilerParams(use_tc_tiling_on_sc=False,
                                           skip_device_barrier=True),
      scratch_types=[
          pltpu.VMEM((NCHUNK1, CH), jnp.int32),
          pltpu.VMEM((NCHUNK1, CH), jnp.int32),
          pltpu.VMEM((NBUF, CH, HALF), jnp.float32),
          pltpu.VMEM_SHARED((N, HALF), jnp.float32),
          pltpu.SemaphoreType.DMA,
          pltpu.SemaphoreType.DMA,
      ],
  )
  def sc_scatter1(table_hbm, src_hbm, dst_hbm, zeros_hbm, out_hbm,
                  src_v, dst_v, buf, acc, gsem, ssem):
    cid = lax.axis_index("c")
    sid = lax.axis_index("s")

    pltpu.sync_copy(zeros_hbm.at[pl.ds(sid * ROWS_PT, ROWS_PT)],
                    acc.at[pl.ds(sid * ROWS_PT, ROWS_PT)])
    pltpu.sync_copy(src_hbm.at[sid], src_v)
    pltpu.sync_copy(dst_hbm.at[sid], dst_v)
    plsc.subcore_barrier()

    def gather(j):
      return pltpu.make_async_copy(
          table_hbm.at[cid].at[src_v.at[j]], buf.at[lax.rem(j, NBUF)], gsem)

    def scat(j):
      return pltpu.make_async_copy(
          buf.at[lax.rem(j, NBUF)], acc.at[dst_v.at[j]], ssem)

    for b in range(NBUF - 1):
      gather(b).start()

    def body(j, carry):
      gather(j).wait()
      scat(j).start(add=True)

      @pl.when(j >= 1)
      def _():
        scat(j - 1).wait()

      @pl.when(j + NBUF - 1 < NCHUNK1)
      def _():
        gather(j + NBUF - 1).start()

      return carry

    lax.fori_loop(0, NCHUNK1, body, 0, unroll=False)
    scat(NCHUNK1 - 1).wait()
    plsc.subcore_barrier()
    pltpu.sync_copy(acc.at[pl.ds(sid * ROWS_PT, ROWS_PT)],
                    out_hbm.at[cid, pl.ds(sid * ROWS_PT, ROWS_PT)])

  return sc_scatter1


def _make_sc_scatter_l2():
  mesh = plsc.VectorSubcoreMesh(core_axis_name="c", subcore_axis_name="s")

  @functools.partial(
      pl.kernel,
      out_type=jax.ShapeDtypeStruct((NC, N, OUT_DIM), jnp.float32),
      mesh=mesh,
      compiler_params=pltpu.CompilerParams(use_tc_tiling_on_sc=False,
                                           skip_device_barrier=True),
      scratch_types=[
          pltpu.VMEM((NCHUNK, CH), jnp.int32),
          pltpu.VMEM((NCHUNK, CH), jnp.int32),
          pltpu.VMEM((NBUF, CH, OUT_DIM), jnp.float32),
          pltpu.VMEM_SHARED((N, OUT_DIM), jnp.float32),
          pltpu.SemaphoreType.DMA,
          pltpu.SemaphoreType.DMA,
      ],
  )
  def sc_scatter2(table_hbm, src_hbm, dst_hbm, zeros_hbm, out_hbm,
                  src_v, dst_v, buf, acc, gsem, ssem):
    cid = lax.axis_index("c")
    sid = lax.axis_index("s")
    wid = sid * NC + cid

    pltpu.sync_copy(zeros_hbm.at[pl.ds(sid * ROWS_PT, ROWS_PT)],
                    acc.at[pl.ds(sid * ROWS_PT, ROWS_PT)])
    pltpu.sync_copy(src_hbm.at[wid], src_v)
    pltpu.sync_copy(dst_hbm.at[wid], dst_v)
    plsc.subcore_barrier()

    def gather(j):
      return pltpu.make_async_copy(
          table_hbm.at[src_v.at[j]], buf.at[lax.rem(j, NBUF)], gsem)

    def scat(j):
      return pltpu.make_async_copy(
          buf.at[lax.rem(j, NBUF)], acc.at[dst_v.at[j]], ssem)

    for b in range(NBUF - 1):
      gather(b).start()

    def body(j, carry):
      gather(j).wait()
      scat(j).start(add=True)

      @pl.when(j >= 1)
      def _():
        scat(j - 1).wait()

      @pl.when(j + NBUF - 1 < NCHUNK)
      def _():
        gather(j + NBUF - 1).start()

      return carry

    lax.fori_loop(0, NCHUNK, body, 0, unroll=False)
    scat(NCHUNK - 1).wait()
    plsc.subcore_barrier()
    pltpu.sync_copy(acc.at[pl.ds(sid * ROWS_PT, ROWS_PT)],
                    out_hbm.at[cid, pl.ds(sid * ROWS_PT, ROWS_PT)])

  return sc_scatter2


DEG_W = 16


def _make_sc_degree():
  mesh = plsc.VectorSubcoreMesh(core_axis_name="c", subcore_axis_name="s")

  @functools.partial(
      pl.kernel,
      out_type=jax.ShapeDtypeStruct((NC, N, DEG_W), jnp.float32),
      mesh=mesh,
      compiler_params=pltpu.CompilerParams(use_tc_tiling_on_sc=False,
                                           skip_device_barrier=True),
      scratch_types=[
          pltpu.VMEM((NCHUNK, CH), jnp.int32),
          pltpu.VMEM((CH, DEG_W), jnp.float32),
          pltpu.VMEM_SHARED((N, DEG_W), jnp.float32),
          pltpu.SemaphoreType.DMA,
      ],
  )
  def sc_degree(ones_hbm, dst_hbm, zeros_hbm, out_hbm, dst_v, buf, acc, sem):
    cid = lax.axis_index("c")
    sid = lax.axis_index("s")
    wid = sid * NC + cid

    @pl.when(sid == 0)
    def _():
      pltpu.sync_copy(zeros_hbm, acc)

    pltpu.sync_copy(dst_hbm.at[wid], dst_v)
    pltpu.sync_copy(ones_hbm, buf)
    plsc.subcore_barrier()

    def body(j, carry):
      pltpu.sync_copy(buf, acc.at[dst_v.at[j]], add=True)
      return carry

    lax.fori_loop(0, NCHUNK, body, 0, unroll=False)
    plsc.subcore_barrier()

    @pl.when(sid == 0)
    def _():
      pltpu.sync_copy(acc, out_hbm.at[cid])

  return sc_degree


_ROW_BLK = 2000
_GRID = N // _ROW_BLK


def _dis_from_degp(degp_blk):
  deg = degp_blk[0, :, 0:1] + degp_blk[1, :, 0:1] + 1.0
  return lax.rsqrt(deg)


def _tc_matmul1(x_ref, w1_ref, h_ref):
  h_ref[...] = jnp.dot(x_ref[...], w1_ref[...],
                       preferred_element_type=jnp.float32)


def _tc_scale1(h_ref, degp_ref, g1_ref):
  dis = _dis_from_degp(degp_ref[...])
  g = h_ref[...] * dis
  g1_ref[0] = g[:, :HALF]
  g1_ref[1] = g[:, HALF:]


def _tc_stage2(s1_ref, g1_ref, degp_ref, b1_ref, w2_ref, g2_ref):
  dis = _dis_from_degp(degp_ref[...])
  t0 = dis * (s1_ref[0] + g1_ref[0]) + b1_ref[0]
  t1 = dis * (s1_ref[1] + g1_ref[1]) + b1_ref[1]
  out1 = jnp.concatenate([t0, t1], axis=1)
  u = _SELU_SCALE * jnp.where(
      out1 > 0, out1, _SELU_ALPHA * (jnp.exp(out1) - 1.0))
  m = jnp.dot(u, w2_ref[...], preferred_element_type=jnp.float32)
  g2_ref[...] = m * dis


def _tc_stage3(s2_ref, g2_ref, degp_ref, b2_ref, out_ref):
  dis = _dis_from_degp(degp_ref[...])
  s = s2_ref[0] + s2_ref[1]
  out_ref[...] = dis * (s + g2_ref[...]) + b2_ref[...]


def _row_spec(d):
  return pl.BlockSpec((_ROW_BLK, d), lambda i: (i, 0))


def _half_spec():
  return pl.BlockSpec((NC, _ROW_BLK, HALF), lambda i: (0, i, 0))


_DEGP_SPEC = pl.BlockSpec((NC, _ROW_BLK, DEG_W), lambda i: (0, i, 0))


def _full_spec(shape):
  nd = len(shape)
  return pl.BlockSpec(shape, lambda i: (0,) * nd)


def kernel(x, edge_index, W1, b1, W2, b2):
  src32 = edge_index[0].astype(jnp.int32)
  dst32 = edge_index[1].astype(jnp.int32)
  src_w = src32.reshape(NW, NCHUNK, CH)
  dst_w = dst32.reshape(NW, NCHUNK, CH)
  src_s = src32.reshape(NS, NCHUNK1, CH)
  dst_s = dst32.reshape(NS, NCHUNK1, CH)

  ones_ch = jnp.ones((CH, DEG_W), jnp.float32)
  zeros_deg = jnp.zeros((N, DEG_W), jnp.float32)
  zeros_half = jnp.zeros((N, HALF), jnp.float32)
  zeros_out = jnp.zeros((N, OUT_DIM), jnp.float32)

  _tc_params = pltpu.CompilerParams(skip_device_barrier=True)
  h = pl.pallas_call(
      _tc_matmul1,
      grid=(_GRID,),
      compiler_params=_tc_params,
      in_specs=[_row_spec(IN_DIM), _full_spec((IN_DIM, HID_DIM))],
      out_specs=_row_spec(HID_DIM),
      out_shape=jax.ShapeDtypeStruct((N, HID_DIM), jnp.float32),
  )(x, W1)

  degp = _make_sc_degree()(ones_ch, dst_w, zeros_deg)

  g1 = pl.pallas_call(
      _tc_scale1,
      grid=(_GRID,),
      compiler_params=_tc_params,
      in_specs=[_row_spec(HID_DIM), _DEGP_SPEC],
      out_specs=_half_spec(),
      out_shape=jax.ShapeDtypeStruct((NC, N, HALF), jnp.float32),
  )(h, degp)

  s1 = _make_sc_scatter_l1()(g1, src_s, dst_s, zeros_half)

  g2 = pl.pallas_call(
      _tc_stage2,
      grid=(_GRID,),
      compiler_params=_tc_params,
      in_specs=[_half_spec(), _half_spec(), _DEGP_SPEC,
                _full_spec((NC, 1, HALF)), _full_spec((HID_DIM, OUT_DIM))],
      out_specs=_row_spec(OUT_DIM),
      out_shape=jax.ShapeDtypeStruct((N, OUT_DIM), jnp.float32),
  )(s1, g1, degp, b1.reshape(NC, 1, HALF), W2)

  s2 = _make_sc_scatter_l2()(g2, src_w, dst_w, zeros_out)

  out = pl.pallas_call(
      _tc_stage3,
      grid=(_GRID,),
      compiler_params=_tc_params,
      in_specs=[pl.BlockSpec((NC, _ROW_BLK, OUT_DIM), lambda i: (0, i, 0)),
                _row_spec(OUT_DIM), _DEGP_SPEC, _full_spec((1, OUT_DIM))],
      out_specs=_row_spec(OUT_DIM),
      out_shape=jax.ShapeDtypeStruct((N, OUT_DIM), jnp.float32),
  )(s2, g2, degp, b2.reshape(1, OUT_DIM))

  return out


# speedup vs baseline: 36.6872x; 1.0076x over previous
import functools

import jax
import jax.numpy as jnp
from jax import lax
from jax.experimental import pallas as pl
from jax.experimental.pallas import tpu as pltpu
from jax.experimental.pallas import tpu_sc as plsc

N = 10000
E = 320000
IN_DIM = 128
HID_DIM = 128
OUT_DIM = 64
HALF = HID_DIM // 2

NC = 2
NS = 16
NW = NC * NS
CH = 125
EPW = E // NW
NCHUNK = EPW // CH
EPS = E // NS
NCHUNK1 = EPS // CH
ROWS_PT = N // NS
NBUF = 4

_SELU_ALPHA = 1.6732632423543772
_SELU_SCALE = 1.0507009873554805


def _make_sc_scatter_l1():
  mesh = plsc.VectorSubcoreMesh(core_axis_name="c", subcore_axis_name="s")

  @functools.partial(
      pl.kernel,
      out_type=jax.ShapeDtypeStruct((NC, N, HALF), jnp.float32),
      mesh=mesh,
      compiler_params=pltpu.CompilerParams(use_tc_tiling_on_sc=False,
                                           skip_device_barrier=True),
      scratch_types=[
          pltpu.VMEM((NCHUNK1, CH), jnp.int32),
          pltpu.VMEM((NCHUNK1, CH), jnp.int32),
          pltpu.VMEM((NBUF, CH, HALF), jnp.float32),
          pltpu.VMEM_SHARED((N, HALF), jnp.float32),
          pltpu.SemaphoreType.DMA,
          pltpu.SemaphoreType.DMA,
      ],
  )
  def sc_scatter1(table_hbm, src_hbm, dst_hbm, zeros_hbm, out_hbm,
                  src_v, dst_v, buf, acc, gsem, ssem):
    cid = lax.axis_index("c")
    sid = lax.axis_index("s")

    pltpu.sync_copy(zeros_hbm.at[pl.ds(sid * ROWS_PT, ROWS_PT)],
                    acc.at[pl.ds(sid * ROWS_PT, ROWS_PT)])
    pltpu.sync_copy(src_hbm.at[sid], src_v)
    pltpu.sync_copy(dst_hbm.at[sid], dst_v)
    plsc.subcore_barrier()

    def gather(j):
      return pltpu.make_async_copy(
          table_hbm.at[cid].at[src_v.at[j]], buf.at[lax.rem(j, NBUF)], gsem)

    def scat(j):
      return pltpu.make_async_copy(
          buf.at[lax.rem(j, NBUF)], acc.at[dst_v.at[j]], ssem)

    for b in range(NBUF - 1):
      gather(b).start()

    def body(j, carry):
      gather(j).wait()
      scat(j).start(add=True)

      @pl.when(j >= 1)
      def _():
        scat(j - 1).wait()

      @pl.when(j + NBUF - 1 < NCHUNK1)
      def _():
        gather(j + NBUF - 1).start()

      return carry

    lax.fori_loop(0, NCHUNK1, body, 0, unroll=False)
    scat(NCHUNK1 - 1).wait()
    plsc.subcore_barrier()
    pltpu.sync_copy(acc.at[pl.ds(sid * ROWS_PT, ROWS_PT)],
                    out_hbm.at[cid, pl.ds(sid * ROWS_PT, ROWS_PT)])

  return sc_scatter1


def _make_sc_scatter_l2():
  mesh = plsc.VectorSubcoreMesh(core_axis_name="c", subcore_axis_name="s")

  @functools.partial(
      pl.kernel,
      out_type=jax.ShapeDtypeStruct((NC, N, OUT_DIM), jnp.float32),
      mesh=mesh,
      compiler_params=pltpu.CompilerParams(use_tc_tiling_on_sc=False,
                                           skip_device_barrier=True),
      scratch_types=[
          pltpu.VMEM((NCHUNK, CH), jnp.int32),
          pltpu.VMEM((NCHUNK, CH), jnp.int32),
          pltpu.VMEM((NBUF, CH, OUT_DIM), jnp.float32),
          pltpu.VMEM_SHARED((N, OUT_DIM), jnp.float32),
          pltpu.SemaphoreType.DMA,
          pltpu.SemaphoreType.DMA,
      ],
  )
  def sc_scatter2(table_hbm, src_hbm, dst_hbm, zeros_hbm, out_hbm,
                  src_v, dst_v, buf, acc, gsem, ssem):
    cid = lax.axis_index("c")
    sid = lax.axis_index("s")
    wid = sid * NC + cid

    pltpu.sync_copy(zeros_hbm.at[pl.ds(sid * ROWS_PT, ROWS_PT)],
                    acc.at[pl.ds(sid * ROWS_PT, ROWS_PT)])
    pltpu.sync_copy(src_hbm.at[wid], src_v)
    pltpu.sync_copy(dst_hbm.at[wid], dst_v)
    plsc.subcore_barrier()

    def gather(j):
      return pltpu.make_async_copy(
          table_hbm.at[src_v.at[j]], buf.at[lax.rem(j, NBUF)], gsem)

    def scat(j):
      return pltpu.make_async_copy(
          buf.at[lax.rem(j, NBUF)], acc.at[dst_v.at[j]], ssem)

    for b in range(NBUF - 1):
      gather(b).start()

    def body(j, carry):
      gather(j).wait()
      scat(j).start(add=True)

      @pl.when(j >= 1)
      def _():
        scat(j - 1).wait()

      @pl.when(j + NBUF - 1 < NCHUNK)
      def _():
        gather(j + NBUF - 1).start()

      return carry

    lax.fori_loop(0, NCHUNK, body, 0, unroll=False)
    scat(NCHUNK - 1).wait()
    plsc.subcore_barrier()
    pltpu.sync_copy(acc.at[pl.ds(sid * ROWS_PT, ROWS_PT)],
                    out_hbm.at[cid, pl.ds(sid * ROWS_PT, ROWS_PT)])

  return sc_scatter2


DEG_W = 16


def _make_sc_degree():
  mesh = plsc.VectorSubcoreMesh(core_axis_name="c", subcore_axis_name="s")

  @functools.partial(
      pl.kernel,
      out_type=jax.ShapeDtypeStruct((NC, N, DEG_W), jnp.float32),
      mesh=mesh,
      compiler_params=pltpu.CompilerParams(use_tc_tiling_on_sc=False,
                                           skip_device_barrier=True),
      scratch_types=[
          pltpu.VMEM((NCHUNK, CH), jnp.int32),
          pltpu.VMEM((CH, DEG_W), jnp.float32),
          pltpu.VMEM_SHARED((N, DEG_W), jnp.float32),
          pltpu.SemaphoreType.DMA,
      ],
  )
  def sc_degree(ones_hbm, dst_hbm, zeros_hbm, out_hbm, dst_v, buf, acc, sem):
    cid = lax.axis_index("c")
    sid = lax.axis_index("s")
    wid = sid * NC + cid

    pltpu.sync_copy(zeros_hbm.at[pl.ds(sid * ROWS_PT, ROWS_PT)],
                    acc.at[pl.ds(sid * ROWS_PT, ROWS_PT)])
    pltpu.sync_copy(dst_hbm.at[wid], dst_v)
    pltpu.sync_copy(ones_hbm, buf)
    plsc.subcore_barrier()

    def scat(j):
      return pltpu.make_async_copy(buf, acc.at[dst_v.at[j]], sem)

    def body(j, carry):
      scat(j).start(add=True)

      @pl.when(j >= 1)
      def _():
        scat(j - 1).wait()

      return carry

    lax.fori_loop(0, NCHUNK, body, 0, unroll=False)
    scat(NCHUNK - 1).wait()
    plsc.subcore_barrier()
    pltpu.sync_copy(acc.at[pl.ds(sid * ROWS_PT, ROWS_PT)],
                    out_hbm.at[cid, pl.ds(sid * ROWS_PT, ROWS_PT)])

  return sc_degree


_ROW_BLK = 2000
_GRID = N // _ROW_BLK


def _dis_from_degp(degp_blk):
  deg = degp_blk[0, :, 0:1] + degp_blk[1, :, 0:1] + 1.0
  return lax.rsqrt(deg)


def _tc_matmul1(x_ref, w1_ref, h_ref):
  h_ref[...] = jnp.dot(x_ref[...], w1_ref[...],
                       preferred_element_type=jnp.float32)


def _tc_scale1(h_ref, degp_ref, g1_ref):
  dis = _dis_from_degp(degp_ref[...])
  g = h_ref[...] * dis
  g1_ref[0] = g[:, :HALF]
  g1_ref[1] = g[:, HALF:]


def _tc_stage2(s1_ref, g1_ref, degp_ref, b1_ref, w2_ref, g2_ref):
  dis = _dis_from_degp(degp_ref[...])
  t0 = dis * (s1_ref[0] + g1_ref[0]) + b1_ref[0]
  t1 = dis * (s1_ref[1] + g1_ref[1]) + b1_ref[1]
  out1 = jnp.concatenate([t0, t1], axis=1)
  u = _SELU_SCALE * jnp.where(
      out1 > 0, out1, _SELU_ALPHA * (jnp.exp(out1) - 1.0))
  m = jnp.dot(u, w2_ref[...], preferred_element_type=jnp.float32)
  g2_ref[...] = m * dis


def _tc_stage3(s2_ref, g2_ref, degp_ref, b2_ref, out_ref):
  dis = _dis_from_degp(degp_ref[...])
  s = s2_ref[0] + s2_ref[1]
  out_ref[...] = dis * (s + g2_ref[...]) + b2_ref[...]


def _row_spec(d):
  return pl.BlockSpec((_ROW_BLK, d), lambda i: (i, 0))


def _half_spec():
  return pl.BlockSpec((NC, _ROW_BLK, HALF), lambda i: (0, i, 0))


_DEGP_SPEC = pl.BlockSpec((NC, _ROW_BLK, DEG_W), lambda i: (0, i, 0))


def _full_spec(shape):
  nd = len(shape)
  return pl.BlockSpec(shape, lambda i: (0,) * nd)


def kernel(x, edge_index, W1, b1, W2, b2):
  src32 = edge_index[0].astype(jnp.int32)
  dst32 = edge_index[1].astype(jnp.int32)
  src_w = src32.reshape(NW, NCHUNK, CH)
  dst_w = dst32.reshape(NW, NCHUNK, CH)
  src_s = src32.reshape(NS, NCHUNK1, CH)
  dst_s = dst32.reshape(NS, NCHUNK1, CH)

  ones_ch = jnp.ones((CH, DEG_W), jnp.float32)
  zeros_deg = jnp.zeros((N, DEG_W), jnp.float32)
  zeros_half = jnp.zeros((N, HALF), jnp.float32)
  zeros_out = jnp.zeros((N, OUT_DIM), jnp.float32)

  _tc_params = pltpu.CompilerParams(skip_device_barrier=True)
  h = pl.pallas_call(
      _tc_matmul1,
      grid=(_GRID,),
      compiler_params=_tc_params,
      in_specs=[_row_spec(IN_DIM), _full_spec((IN_DIM, HID_DIM))],
      out_specs=_row_spec(HID_DIM),
      out_shape=jax.ShapeDtypeStruct((N, HID_DIM), jnp.float32),
  )(x, W1)

  degp = _make_sc_degree()(ones_ch, dst_w, zeros_deg)

  g1 = pl.pallas_call(
      _tc_scale1,
      grid=(_GRID,),
      compiler_params=_tc_params,
      in_specs=[_row_spec(HID_DIM), _DEGP_SPEC],
      out_specs=_half_spec(),
      out_shape=jax.ShapeDtypeStruct((NC, N, HALF), jnp.float32),
  )(h, degp)

  s1 = _make_sc_scatter_l1()(g1, src_s, dst_s, zeros_half)

  g2 = pl.pallas_call(
      _tc_stage2,
      grid=(_GRID,),
      compiler_params=_tc_params,
      in_specs=[_half_spec(), _half_spec(), _DEGP_SPEC,
                _full_spec((NC, 1, HALF)), _full_spec((HID_DIM, OUT_DIM))],
      out_specs=_row_spec(OUT_DIM),
      out_shape=jax.ShapeDtypeStruct((N, OUT_DIM), jnp.float32),
  )(s1, g1, degp, b1.reshape(NC, 1, HALF), W2)

  s2 = _make_sc_scatter_l2()(g2, src_w, dst_w, zeros_out)

  out = pl.pallas_call(
      _tc_stage3,
      grid=(_GRID,),
      compiler_params=_tc_params,
      in_specs=[pl.BlockSpec((NC, _ROW_BLK, OUT_DIM), lambda i: (0, i, 0)),
                _row_spec(OUT_DIM), _DEGP_SPEC, _full_spec((1, OUT_DIM))],
      out_specs=_row_spec(OUT_DIM),
      out_shape=jax.ShapeDtypeStruct((N, OUT_DIM), jnp.float32),
  )(s2, g2, degp, b2.reshape(1, OUT_DIM))

  return out
